# baseline (speedup 1.0000x reference)
import functools

import jax
import jax.numpy as jnp
from jax import lax
from jax.experimental import pallas as pl
from jax.experimental.pallas import tpu as pltpu
from jax.experimental.pallas import tpu_sc as plsc

_NUM_CORES = 2
_NUM_SUBCORES = 16
_NW = _NUM_CORES * _NUM_SUBCORES


@functools.cache
def _make_gather(batch: int, hidden: int):
    assert batch % _NW == 0
    b_per_w = batch // _NW
    mesh = plsc.VectorSubcoreMesh(core_axis_name="c", subcore_axis_name="s")

    @functools.partial(
        pl.kernel,
        mesh=mesh,
        out_type=jax.ShapeDtypeStruct((batch, hidden), jnp.float32),
        scratch_types=[
            pltpu.VMEM((b_per_w,), jnp.int32),
            pltpu.VMEM((b_per_w, hidden), jnp.float32),
            pltpu.SemaphoreType.DMA,
        ],
    )
    def gather_kernel(idx_hbm, table_hbm, out_hbm, idx_v, rows_v, sem):
        wid = lax.axis_index("s") * _NUM_CORES + lax.axis_index("c")
        base = wid * b_per_w
        pltpu.sync_copy(idx_hbm.at[pl.ds(base, b_per_w)], idx_v)

        @plsc.parallel_loop(0, b_per_w // 16, 1, unroll=4)
        def body(g):
            vec = idx_v[pl.ds(g * 16, 16)]
            for j in range(16):
                pltpu.async_copy(
                    table_hbm.at[pl.ds(vec[j], 1)],
                    rows_v.at[pl.ds(g * 16 + j, 1)], sem)
        pltpu.make_async_copy(
            table_hbm.at[pl.ds(0, b_per_w)], rows_v, sem).wait()
        pltpu.sync_copy(rows_v, out_hbm.at[pl.ds(base, b_per_w)])

    return gather_kernel


def kernel(labels, train, table):
    del train
    idx = labels.astype(jnp.int32)
    return _make_gather(idx.shape[0], table.shape[1])(idx, table)


# speedup vs baseline: 1.0048x; 1.0048x over previous
import functools

import jax
import jax.numpy as jnp
from jax import lax
from jax.experimental import pallas as pl
from jax.experimental.pallas import tpu as pltpu
from jax.experimental.pallas import tpu_sc as plsc

_NUM_CORES = 2
_NUM_SUBCORES = 16
_NW = _NUM_CORES * _NUM_SUBCORES


@functools.cache
def _make_gather(batch: int, hidden: int):
    assert batch % _NW == 0
    b_per_w = batch // _NW
    mesh = plsc.VectorSubcoreMesh(core_axis_name="c", subcore_axis_name="s")

    @functools.partial(
        pl.kernel,
        mesh=mesh,
        out_type=jax.ShapeDtypeStruct((batch, hidden), jnp.float32),
        scratch_types=[
            pltpu.VMEM((b_per_w,), jnp.int32),
            pltpu.VMEM((b_per_w, hidden), jnp.float32),
            pltpu.SemaphoreType.DMA,
            pltpu.SemaphoreType.DMA,
            pltpu.SemaphoreType.DMA,
            pltpu.SemaphoreType.DMA,
        ],
    )
    def gather_kernel(idx_hbm, table_hbm, out_hbm, idx_v, rows_v, s0, s1, s2, s3):
        wid = lax.axis_index("s") * _NUM_CORES + lax.axis_index("c")
        base = wid * b_per_w
        sems = (s0, s1, s2, s3)
        pltpu.sync_copy(idx_hbm.at[pl.ds(base, b_per_w)], idx_v)

        @plsc.parallel_loop(0, b_per_w // 16, 1, unroll=4)
        def body(g):
            vec = idx_v[pl.ds(g * 16, 16)]
            for j in range(16):
                pltpu.async_copy(
                    table_hbm.at[pl.ds(vec[j], 1)],
                    rows_v.at[pl.ds(g * 16 + j, 1)], sems[j % 4])
        for q in range(4):
            pltpu.make_async_copy(
                table_hbm.at[pl.ds(0, b_per_w // 4)],
                rows_v.at[pl.ds(0, b_per_w // 4)], sems[q]).wait()
        pltpu.sync_copy(rows_v, out_hbm.at[pl.ds(base, b_per_w)])

    return gather_kernel


def kernel(labels, train, table):
    del train
    idx = labels.astype(jnp.int32)
    return _make_gather(idx.shape[0], table.shape[1])(idx, table)
